# baseline (speedup 1.0000x reference)
import functools

import jax
import jax.numpy as jnp
from jax import lax
from jax.experimental import pallas as pl
from jax.experimental.pallas import tpu as pltpu
from jax.experimental.pallas import tpu_sc as plsc

B = 16384
D = 64
NC = 2
NS = 16
NW = NC * NS
BPW = B // NW
CHUNK = 128
NCHUNK = BPW // CHUNK
GPC = CHUNK // 16

_mesh = plsc.VectorSubcoreMesh(core_axis_name="c", subcore_axis_name="s")


@functools.partial(
    pl.kernel,
    mesh=_mesh,
    out_type=jax.ShapeDtypeStruct((B,), jnp.float32),
    compiler_params=pltpu.CompilerParams(needs_layout_passes=False),
    scratch_types=[
        pltpu.VMEM((BPW,), jnp.int32),
        pltpu.VMEM((BPW,), jnp.int32),
        pltpu.VMEM((BPW,), jnp.int32),
        pltpu.VMEM((BPW // 2, 128), jnp.float32),
        pltpu.VMEM((BPW // 2, 128), jnp.float32),
        pltpu.VMEM((BPW // 2, 128), jnp.float32),
        pltpu.VMEM((BPW,), jnp.float32),
        pltpu.SemaphoreType.DMA((NCHUNK,)),
    ],
)
def _bt_norm_kernel(i_hbm, j_hbm, k_hbm, u_hbm, v_hbm, out_hbm,
                    iv, jv, kv, uv, vjv, vkv, outv, sems):
    wid = lax.axis_index("s") * NC + lax.axis_index("c")
    base = wid * BPW
    pltpu.sync_copy(i_hbm.at[pl.ds(base, BPW)], iv)
    pltpu.sync_copy(j_hbm.at[pl.ds(base, BPW)], jv)
    pltpu.sync_copy(k_hbm.at[pl.ds(base, BPW)], kv)

    def issue16(t, carry):
        ivec = iv[pl.ds(t * 16, 16)]
        jvec = jv[pl.ds(t * 16, 16)]
        kvec = kv[pl.ds(t * 16, 16)]
        c = t // (CHUNK // 16)
        for l in range(16):
            dst = (t * 8 + (l // 2), pl.ds((l % 2) * D, D))
            pltpu.async_copy(u_hbm.at[ivec[l]], uv.at[dst], sems.at[c])
            pltpu.async_copy(v_hbm.at[jvec[l]], vjv.at[dst], sems.at[c])
            pltpu.async_copy(v_hbm.at[kvec[l]], vkv.at[dst], sems.at[c])
        return carry

    def issue_chunk(c):
        lax.fori_loop(c * (CHUNK // 16), (c + 1) * (CHUNK // 16), issue16, 0)

    issue_chunk(0)

    lane = lax.iota(jnp.int32, 16)
    halfrow = lax.shift_right_logical(lane, 1)
    colbase = (lane & 1) * D

    def group(g, carry):
        rows2 = g * 8 + halfrow
        accj = jnp.zeros((16,), jnp.float32)
        acck = jnp.zeros((16,), jnp.float32)
        for d in range(D):
            col = colbase + ((lane + d) & (D - 1))
            uval = plsc.load_gather(uv, [rows2, col])
            jval = plsc.load_gather(vjv, [rows2, col])
            kval = plsc.load_gather(vkv, [rows2, col])
            dj = uval - jval
            dk = uval - kval
            accj = accj + dj * dj
            acck = acck + dk * dk
        x = accj - acck
        outv[pl.ds(g * 16, 16)] = 1.0 / (1.0 + jnp.exp(x))
        return carry

    for c in range(NCHUNK):
        if c + 1 < NCHUNK:
            issue_chunk(c + 1)
        def drain(t, carry):
            pltpu.make_async_copy(i_hbm.at[pl.ds(0, BPW)], iv, sems.at[c]).wait()
            return carry
        lax.fori_loop(0, (3 * CHUNK * D) // BPW, drain, 0)
        lax.fori_loop(c * GPC, (c + 1) * GPC, group, 0)

    pltpu.sync_copy(outv, out_hbm.at[pl.ds(base, BPW)])


def kernel(i, j, k, u_weight, v_weight):
    return _bt_norm_kernel(
        i.astype(jnp.int32), j.astype(jnp.int32), k.astype(jnp.int32),
        u_weight, v_weight)


# speedup vs baseline: 1.0133x; 1.0133x over previous
import functools

import jax
import jax.numpy as jnp
from jax import lax
from jax.experimental import pallas as pl
from jax.experimental.pallas import tpu as pltpu
from jax.experimental.pallas import tpu_sc as plsc

B = 16384
D = 64
NC = 2
NS = 16
NW = NC * NS
BPW = B // NW
CHUNK = 128
NCHUNK = BPW // CHUNK
GPC = CHUNK // 16

_mesh = plsc.VectorSubcoreMesh(core_axis_name="c", subcore_axis_name="s")


@functools.partial(
    pl.kernel,
    mesh=_mesh,
    out_type=jax.ShapeDtypeStruct((B,), jnp.float32),
    compiler_params=pltpu.CompilerParams(needs_layout_passes=False),
    scratch_types=[
        pltpu.VMEM((BPW,), jnp.int32),
        pltpu.VMEM((BPW,), jnp.int32),
        pltpu.VMEM((BPW,), jnp.int32),
        pltpu.VMEM((BPW // 2, 128), jnp.float32),
        pltpu.VMEM((BPW // 2, 128), jnp.float32),
        pltpu.VMEM((BPW // 2, 128), jnp.float32),
        pltpu.VMEM((BPW,), jnp.float32),
        pltpu.SemaphoreType.DMA((NCHUNK,)),
    ],
)
def _bt_norm_kernel(i_hbm, j_hbm, k_hbm, u_hbm, v_hbm, out_hbm,
                    iv, jv, kv, uv, vjv, vkv, outv, sems):
    wid = lax.axis_index("s") * NC + lax.axis_index("c")
    base = wid * BPW
    pltpu.sync_copy(i_hbm.at[pl.ds(base, BPW)], iv)
    pltpu.sync_copy(j_hbm.at[pl.ds(base, BPW)], jv)
    pltpu.sync_copy(k_hbm.at[pl.ds(base, BPW)], kv)

    def issue16(t, carry):
        ivec = iv[pl.ds(t * 16, 16)]
        jvec = jv[pl.ds(t * 16, 16)]
        kvec = kv[pl.ds(t * 16, 16)]
        c = t // (CHUNK // 16)
        for l in range(16):
            dst = (t * 8 + (l // 2), pl.ds((l % 2) * D, D))
            pltpu.async_copy(u_hbm.at[ivec[l]], uv.at[dst], sems.at[c])
            pltpu.async_copy(v_hbm.at[jvec[l]], vjv.at[dst], sems.at[c])
            pltpu.async_copy(v_hbm.at[kvec[l]], vkv.at[dst], sems.at[c])
        return carry

    lax.fori_loop(0, BPW // 16, issue16, 0)

    lane = lax.iota(jnp.int32, 16)
    halfrow = lax.shift_right_logical(lane, 1)
    colbase = (lane & 1) * D

    def group(g, carry):
        rows2 = g * 8 + halfrow
        accj = jnp.zeros((16,), jnp.float32)
        acck = jnp.zeros((16,), jnp.float32)
        for d in range(D):
            col = colbase + ((lane + d) & (D - 1))
            uval = plsc.load_gather(uv, [rows2, col])
            jval = plsc.load_gather(vjv, [rows2, col])
            kval = plsc.load_gather(vkv, [rows2, col])
            dj = uval - jval
            dk = uval - kval
            accj = accj + dj * dj
            acck = acck + dk * dk
        x = accj - acck
        outv[pl.ds(g * 16, 16)] = 1.0 / (1.0 + jnp.exp(x))
        return carry

    for c in range(NCHUNK):
        def drain(t, carry):
            pltpu.make_async_copy(i_hbm.at[pl.ds(0, BPW)], iv, sems.at[c]).wait()
            return carry
        lax.fori_loop(0, (3 * CHUNK * D) // BPW, drain, 0)
        lax.fori_loop(c * GPC, (c + 1) * GPC, group, 0)

    pltpu.sync_copy(outv, out_hbm.at[pl.ds(base, BPW)])


def kernel(i, j, k, u_weight, v_weight):
    return _bt_norm_kernel(
        i.astype(jnp.int32), j.astype(jnp.int32), k.astype(jnp.int32),
        u_weight, v_weight)
